# Initial kernel scaffold; baseline (speedup 1.0000x reference)
import jax
import jax.numpy as jnp
from jax.experimental import pallas as pl


def kernel(x, W_route, b_route, W_noise, b_noise, noise_raw):
    raise NotImplementedError("write your pallas kernel here")


# speedup vs baseline: 3.1751x; 3.1751x over previous
import jax
import jax.numpy as jnp
from jax.experimental import pallas as pl

_T, _D, _E, _K = 8192, 4096, 64, 8
_BT = 256


def _router_body(x_ref, wr_ref, br_ref, wn_ref, bn_ref, nz_ref, out_ref, idx_ref):
    x = x_ref[...]
    dn = (((1,), (1,)), ((), ()))
    logits = jax.lax.dot_general(
        x, wr_ref[...], dn, preferred_element_type=jnp.float32,
        precision=jax.lax.Precision.DEFAULT) + br_ref[...]
    nlog = jax.lax.dot_general(
        x, wn_ref[...], dn, preferred_element_type=jnp.float32,
        precision=jax.lax.Precision.DEFAULT) + bn_ref[...]
    sp = jnp.maximum(nlog, 0.0) + jnp.log1p(jnp.exp(-jnp.abs(nlog)))
    noisy = logits + nz_ref[...] * sp

    neg_inf = jnp.float32(-jnp.inf)
    iota = jax.lax.broadcasted_iota(jnp.int32, (_BT, _E), 1)
    work = noisy
    sel = jnp.zeros((_BT, _E), jnp.bool_)
    idx_cols = []
    m0 = None
    for k in range(_K):
        m = jnp.max(work, axis=1, keepdims=True)
        if k == 0:
            m0 = m
        idx = jnp.min(jnp.where(work == m, iota, _E), axis=1, keepdims=True)
        chosen = iota == idx
        sel = jnp.logical_or(sel, chosen)
        work = jnp.where(chosen, neg_inf, work)
        idx_cols.append(idx)
    idx_ref[...] = jnp.concatenate(idx_cols, axis=1)
    e = jnp.where(sel, jnp.exp(noisy - m0), 0.0)
    out_ref[...] = e / jnp.sum(e, axis=1, keepdims=True)


def kernel(x, W_route, b_route, W_noise, b_noise, noise_raw):
    br = b_route.reshape(1, _E)
    bn = b_noise.reshape(1, _E)
    grid = (_T // _BT,)
    out, idx = pl.pallas_call(
        _router_body,
        grid=grid,
        in_specs=[
            pl.BlockSpec((_BT, _D), lambda i: (i, 0)),
            pl.BlockSpec((_E, _D), lambda i: (0, 0)),
            pl.BlockSpec((1, _E), lambda i: (0, 0)),
            pl.BlockSpec((_E, _D), lambda i: (0, 0)),
            pl.BlockSpec((1, _E), lambda i: (0, 0)),
            pl.BlockSpec((_BT, _E), lambda i: (i, 0)),
        ],
        out_specs=[
            pl.BlockSpec((_BT, _E), lambda i: (i, 0)),
            pl.BlockSpec((_BT, _K), lambda i: (i, 0)),
        ],
        out_shape=[
            jax.ShapeDtypeStruct((_T, _E), jnp.float32),
            jax.ShapeDtypeStruct((_T, _K), jnp.int32),
        ],
    )(x, W_route, br, W_noise, bn, noise_raw)
    return (out, idx)
